# baseline (speedup 1.0000x reference)
import jax
import jax.numpy as jnp
import numpy as np
from jax.experimental import pallas as pl
from jax.experimental.pallas import tpu as pltpu

_PARTS = 8
_MU = 0.0001
_TWO_PI = 2.0 * np.pi


def _sc_kernel(tr_ref, nt_ref, sc_ref, dir_ref):
    nt = nt_ref[...]
    tr = tr_ref[...]

    fx = nt[:, :, 0]
    fy = nt[:, :, 1]
    lx = nt[:, :, 14]
    ly = nt[:, :, 15]
    nei_sum = jnp.sum(nt, axis=2)

    tx0 = tr[:, 0:1]
    ty0 = tr[:, 1:2]
    tx1 = tr[:, 14:15]
    ty1 = tr[:, 15:16]

    vx = lx - fx
    vy = ly - fy
    nei_len = jnp.sqrt(vx * vx + vy * vy)
    ovx = tx1 - tx0
    ovy = ty1 - ty0
    obs_len = jnp.sqrt(ovx * ovx + ovy * ovy)
    f_speed = (nei_len + _MU) / (obs_len + _MU)

    px = lx - tx1
    py = ly - ty1
    f_dist = jnp.sqrt(px * px + py * py)
    f_dir = jnp.arctan2(py, px)
    f_dir = jnp.mod(f_dir, _TWO_PI)

    ang = (f_dir / (_TWO_PI / _PARTS)).astype(jnp.int32)
    valid = nei_sum != 0.0
    ang = jnp.where(valid, ang, -1)

    dir_ref[...] = f_dir

    cols = []
    for a in range(_PARTS):
        m = (ang == a).astype(jnp.float32)
        n = jnp.sum(m, axis=1, keepdims=True) + 0.0001
        sp = jnp.sum(f_speed * m, axis=1, keepdims=True) / n
        di = jnp.sum(f_dist * m, axis=1, keepdims=True) / n
        dr = jnp.sum(f_dir * m, axis=1, keepdims=True) / n
        cols.extend([sp, di, dr])
    sc_ref[...] = jnp.concatenate(cols, axis=1)


def kernel(trajs, nei_trajs):
    B = trajs.shape[0]
    tr = trajs.reshape(B, 16)
    nt = nei_trajs.reshape(B, 64, 16)
    BB = 128
    grid = (B // BB,)
    sc24, f_dir = pl.pallas_call(
        _sc_kernel,
        grid=grid,
        in_specs=[
            pl.BlockSpec((BB, 16), lambda i: (i, 0)),
            pl.BlockSpec((BB, 64, 16), lambda i: (i, 0, 0)),
        ],
        out_specs=[
            pl.BlockSpec((BB, 24), lambda i: (i, 0)),
            pl.BlockSpec((BB, 64), lambda i: (i, 0)),
        ],
        out_shape=[
            jax.ShapeDtypeStruct((B, 24), jnp.float32),
            jax.ShapeDtypeStruct((B, 64), jnp.float32),
        ],
    )(tr, nt)
    return sc24.reshape(B, 8, 3), f_dir


# speedup vs baseline: 17.9112x; 17.9112x over previous
import jax
import jax.numpy as jnp
import numpy as np
from jax.experimental import pallas as pl

_PARTS = 8
_MU = 0.0001
_TWO_PI = 2.0 * np.pi


def _sc_kernel(tr_ref, nt_ref, spd_ref, dst_ref, drc_ref, fdir_ref):
    x = nt_ref[...]
    t = tr_ref[...]
    BB = x.shape[0]

    xT = jnp.transpose(x, (1, 0))
    tT = jnp.transpose(t, (1, 0))
    x3 = xT.reshape(64, 16, BB)

    nei_sum = jnp.sum(x3, axis=1)
    fx = x3[:, 0, :]
    fy = x3[:, 1, :]
    lx = x3[:, 14, :]
    ly = x3[:, 15, :]

    tx0 = tT[0:1]
    ty0 = tT[1:2]
    tx1 = tT[14:15]
    ty1 = tT[15:16]

    vx = lx - fx
    vy = ly - fy
    nei_len = jnp.sqrt(vx * vx + vy * vy)
    ovx = tx1 - tx0
    ovy = ty1 - ty0
    obs_len = jnp.sqrt(ovx * ovx + ovy * ovy)
    f_speed = (nei_len + _MU) / (obs_len + _MU)

    px = lx - tx1
    py = ly - ty1
    f_dist = jnp.sqrt(px * px + py * py)
    f_dir = jnp.arctan2(py, px)
    f_dir = jnp.mod(f_dir, _TWO_PI)

    ang = (f_dir / (_TWO_PI / _PARTS)).astype(jnp.int32)
    ang = jnp.where(nei_sum != 0.0, ang, -1)

    fdir_ref[...] = jnp.transpose(f_dir, (1, 0))

    rn, rs, rd, rr = [], [], [], []
    for a in range(_PARTS):
        m = (ang == a).astype(jnp.float32)
        rn.append(jnp.sum(m, axis=0, keepdims=True))
        rs.append(jnp.sum(f_speed * m, axis=0, keepdims=True))
        rd.append(jnp.sum(f_dist * m, axis=0, keepdims=True))
        rr.append(jnp.sum(f_dir * m, axis=0, keepdims=True))
    n8 = jnp.concatenate(rn, axis=0) + 0.0001
    spd_ref[...] = jnp.transpose(jnp.concatenate(rs, axis=0) / n8, (1, 0))
    dst_ref[...] = jnp.transpose(jnp.concatenate(rd, axis=0) / n8, (1, 0))
    drc_ref[...] = jnp.transpose(jnp.concatenate(rr, axis=0) / n8, (1, 0))


def kernel(trajs, nei_trajs):
    B = trajs.shape[0]
    tr = trajs.reshape(B, 16)
    nt = nei_trajs.reshape(B, 1024)
    BB = 512
    grid = (B // BB,)
    spd, dst, drc, f_dir = pl.pallas_call(
        _sc_kernel,
        grid=grid,
        in_specs=[
            pl.BlockSpec((BB, 16), lambda i: (i, 0)),
            pl.BlockSpec((BB, 1024), lambda i: (i, 0)),
        ],
        out_specs=[
            pl.BlockSpec((BB, 8), lambda i: (i, 0)),
            pl.BlockSpec((BB, 8), lambda i: (i, 0)),
            pl.BlockSpec((BB, 8), lambda i: (i, 0)),
            pl.BlockSpec((BB, 64), lambda i: (i, 0)),
        ],
        out_shape=[
            jax.ShapeDtypeStruct((B, 8), jnp.float32),
            jax.ShapeDtypeStruct((B, 8), jnp.float32),
            jax.ShapeDtypeStruct((B, 8), jnp.float32),
            jax.ShapeDtypeStruct((B, 64), jnp.float32),
        ],
    )(tr, nt)
    return jnp.stack([spd, dst, drc], axis=2), f_dir


# speedup vs baseline: 33.7076x; 1.8819x over previous
import jax
import jax.numpy as jnp
import numpy as np
from jax.experimental import pallas as pl

_PARTS = 8
_MU = 0.0001
_TWO_PI = 2.0 * np.pi
_N = 64
_F = 16


def _build_select() -> np.ndarray:
    s = np.zeros((_N * _F, 5 * _N), dtype=np.float32)
    for n in range(_N):
        s[_F * n + 0, 0 * _N + n] = 1.0
        s[_F * n + 1, 1 * _N + n] = 1.0
        s[_F * n + 14, 2 * _N + n] = 1.0
        s[_F * n + 15, 3 * _N + n] = 1.0
        s[_F * n: _F * (n + 1), 4 * _N + n] = 1.0
    return s


def _build_reduce() -> np.ndarray:
    p = np.zeros((32 * _N, 32), dtype=np.float32)
    for a in range(_PARTS):
        for q in range(4):
            k = a * 4 + q
            p[k * _N: (k + 1) * _N, q * _PARTS + a] = 1.0
    return p


def _sc_kernel(tr_ref, nt_ref, sel_ref, red_ref, spd_ref, dst_ref, drc_ref,
               fdir_ref):
    x = nt_ref[...]
    t = tr_ref[...]

    feat = jax.lax.dot_general(
        x, sel_ref[...], (((1,), (0,)), ((), ())),
        preferred_element_type=jnp.float32)
    fx = feat[:, 0 * _N:1 * _N]
    fy = feat[:, 1 * _N:2 * _N]
    lx = feat[:, 2 * _N:3 * _N]
    ly = feat[:, 3 * _N:4 * _N]
    nei_sum = feat[:, 4 * _N:5 * _N]

    tx0 = t[:, 0:1]
    ty0 = t[:, 1:2]
    tx1 = t[:, 14:15]
    ty1 = t[:, 15:16]

    vx = lx - fx
    vy = ly - fy
    nei_len = jnp.sqrt(vx * vx + vy * vy)
    ovx = tx1 - tx0
    ovy = ty1 - ty0
    obs_len = jnp.sqrt(ovx * ovx + ovy * ovy)
    f_speed = (nei_len + _MU) / (obs_len + _MU)

    px = lx - tx1
    py = ly - ty1
    f_dist = jnp.sqrt(px * px + py * py)
    f_dir = jnp.arctan2(py, px)
    f_dir = jnp.mod(f_dir, _TWO_PI)

    ang = (f_dir / (_TWO_PI / _PARTS)).astype(jnp.int32)
    ang = jnp.where(nei_sum != 0.0, ang, -1)

    fdir_ref[...] = f_dir

    blocks = []
    one = jnp.ones_like(f_dir)
    zero = jnp.zeros_like(f_dir)
    for a in range(_PARTS):
        m = ang == a
        blocks.append(jnp.where(m, one, zero))
        blocks.append(jnp.where(m, f_speed, zero))
        blocks.append(jnp.where(m, f_dist, zero))
        blocks.append(jnp.where(m, f_dir, zero))
    masked = jnp.concatenate(blocks, axis=1)
    sums = jax.lax.dot_general(
        masked, red_ref[...], (((1,), (0,)), ((), ())),
        preferred_element_type=jnp.float32)

    n8 = sums[:, 0:8] + 0.0001
    spd_ref[...] = sums[:, 8:16] / n8
    dst_ref[...] = sums[:, 16:24] / n8
    drc_ref[...] = sums[:, 24:32] / n8


def kernel(trajs, nei_trajs):
    B = trajs.shape[0]
    tr = trajs.reshape(B, 16)
    nt = nei_trajs.reshape(B, _N * _F)
    sel = jnp.asarray(_build_select())
    red = jnp.asarray(_build_reduce())
    BB = 512
    grid = (B // BB,)
    spd, dst, drc, f_dir = pl.pallas_call(
        _sc_kernel,
        grid=grid,
        in_specs=[
            pl.BlockSpec((BB, 16), lambda i: (i, 0)),
            pl.BlockSpec((BB, _N * _F), lambda i: (i, 0)),
            pl.BlockSpec((_N * _F, 5 * _N), lambda i: (0, 0)),
            pl.BlockSpec((32 * _N, 32), lambda i: (0, 0)),
        ],
        out_specs=[
            pl.BlockSpec((BB, 8), lambda i: (i, 0)),
            pl.BlockSpec((BB, 8), lambda i: (i, 0)),
            pl.BlockSpec((BB, 8), lambda i: (i, 0)),
            pl.BlockSpec((BB, _N), lambda i: (i, 0)),
        ],
        out_shape=[
            jax.ShapeDtypeStruct((B, 8), jnp.float32),
            jax.ShapeDtypeStruct((B, 8), jnp.float32),
            jax.ShapeDtypeStruct((B, 8), jnp.float32),
            jax.ShapeDtypeStruct((B, _N), jnp.float32),
        ],
    )(tr, nt, sel, red)
    return jnp.stack([spd, dst, drc], axis=2), f_dir
